# baseline (speedup 1.0000x reference)
import functools

import jax
import jax.numpy as jnp
from jax.experimental import pallas as pl
from jax.experimental.pallas import tpu as pltpu

_ROWS = 8192
_COLS = 128
_BLOCK_ROWS = 512


def _tail_body(flag_ref, a0, b0, w0, r0, a1, b1, w1, r1, m, out_ref):
    v0 = a0[...] + w0[...] * (b0[...] - a0[...])
    v0 = jnp.where(flag_ref[0] > 0, r0[...], v0)
    v1 = a1[...] + w1[...] * (b1[...] - a1[...])
    v1 = jnp.where(flag_ref[1] > 0, r1[...], v1)
    diff = jnp.where(m[...] > 0, v0 - v1, 0.0)
    part = jnp.reshape(jnp.sum(diff * diff), (1, 1))

    @pl.when(pl.program_id(0) == 0)
    def _():
        out_ref[...] = jnp.zeros_like(out_ref)

    out_ref[...] += part


def _interp_operands(sorted_key, base, count, max_len, i):
    n = sorted_key.shape[0]
    cf = (count - 1).astype(jnp.float32)
    step = cf / (max_len - 1).astype(jnp.float32)
    pos = i.astype(jnp.float32) * step
    pos = jnp.where(i == max_len - 1, cf, pos)
    k = jnp.clip(jnp.floor(pos).astype(jnp.int32) + 1, 1, count - 1)
    w = pos - (k - 1).astype(jnp.float32)

    def val(idx):
        idx = jnp.clip(base + idx, 0, n - 1)
        bits = jnp.take(sorted_key, idx, mode="clip") & 0x3FFFFFFF
        return jax.lax.bitcast_convert_type(bits, jnp.float32)

    return val(k - 1), val(k), w, val(i)


def kernel(y_pred, s, y_gt, pct_a, pct_b):
    n = y_pred.shape[0]
    bits = jax.lax.bitcast_convert_type(y_pred, jnp.int32)
    key = bits | (s << 30)
    sorted_key = jnp.sort(key)
    count1 = jnp.sum(s, dtype=jnp.int32)
    count0 = n - count1
    max_len = jnp.maximum(count0, count1)

    n_pad = _ROWS * _COLS
    i = jnp.arange(n_pad, dtype=jnp.int32)
    a0, b0, w0, r0 = _interp_operands(sorted_key, 0, count0, max_len, i)
    a1, b1, w1, r1 = _interp_operands(sorted_key, count0, count1, max_len, i)
    m = (i < max_len).astype(jnp.float32)
    flags = jnp.stack([
        (count0 == max_len).astype(jnp.float32),
        (count1 == max_len).astype(jnp.float32),
    ])

    shp = (_ROWS, _COLS)
    ops = [x.reshape(shp) for x in (a0, b0, w0, r0, a1, b1, w1, r1, m)]

    grid = _ROWS // _BLOCK_ROWS
    blk = pl.BlockSpec((_BLOCK_ROWS, _COLS), lambda g: (g, 0))
    out = pl.pallas_call(
        _tail_body,
        grid=(grid,),
        in_specs=[pl.BlockSpec(memory_space=pltpu.SMEM)] + [blk] * 9,
        out_specs=pl.BlockSpec((1, 1), lambda g: (0, 0)),
        out_shape=jax.ShapeDtypeStruct((1, 1), jnp.float32),
    )(flags, *ops)

    reg_loss = out[0, 0]
    z = jnp.zeros((1,), dtype=jnp.float32)
    return (reg_loss, z, z, z)


# speedup vs baseline: 1.2516x; 1.2516x over previous
import functools

import jax
import jax.numpy as jnp
from jax.experimental import pallas as pl
from jax.experimental.pallas import tpu as pltpu

_ROWS = 8192
_COLS = 128
_BLOCK_ROWS = 512


def _tail_body(flag_ref, a0, b0, w0, r0, a1, b1, w1, r1, m, out_ref):
    v0 = a0[...] + w0[...] * (b0[...] - a0[...])
    v0 = jnp.where(flag_ref[0] > 0, r0[...], v0)
    v1 = a1[...] + w1[...] * (b1[...] - a1[...])
    v1 = jnp.where(flag_ref[1] > 0, r1[...], v1)
    diff = jnp.where(m[...] > 0, v0 - v1, 0.0)
    part = jnp.reshape(jnp.sum(diff * diff), (1, 1))

    @pl.when(pl.program_id(0) == 0)
    def _():
        out_ref[...] = jnp.zeros_like(out_ref)

    out_ref[...] += part


def _interp_operands(sorted_key, base, count, max_len, i):
    n = sorted_key.shape[0]
    cf = (count - 1).astype(jnp.float32)
    step = cf / (max_len - 1).astype(jnp.float32)
    pos = i.astype(jnp.float32) * step
    pos = jnp.where(i == max_len - 1, cf, pos)
    k = jnp.clip(jnp.floor(pos).astype(jnp.int32) + 1, 1, count - 1)
    w = pos - (k - 1).astype(jnp.float32)

    def val(idx):
        idx = jnp.clip(base + idx, 0, n - 1)
        bits = jnp.take(sorted_key, idx, mode="clip") & 0x3FFFFFFF
        return jax.lax.bitcast_convert_type(bits, jnp.float32)

    return val(k - 1), val(k), w


def kernel(y_pred, s, y_gt, pct_a, pct_b):
    n = y_pred.shape[0]
    bits = jax.lax.bitcast_convert_type(y_pred, jnp.int32)
    key = bits | (s << 30)
    sorted_key = jnp.sort(key)
    count1 = jnp.sum(s, dtype=jnp.int32)
    count0 = n - count1
    max_len = jnp.maximum(count0, count1)

    n_pad = _ROWS * _COLS
    i = jnp.arange(n_pad, dtype=jnp.int32)
    a0, b0, w0 = _interp_operands(sorted_key, 0, count0, max_len, i)
    a1, b1, w1 = _interp_operands(sorted_key, count0, count1, max_len, i)
    vals_ext = jax.lax.bitcast_convert_type(
        jnp.pad(sorted_key, (0, n_pad), mode="edge") & 0x3FFFFFFF,
        jnp.float32)
    r0 = vals_ext[:n_pad]
    r1 = jax.lax.dynamic_slice(vals_ext, (count0,), (n_pad,))
    m = (i < max_len).astype(jnp.float32)
    flags = jnp.stack([
        (count0 == max_len).astype(jnp.float32),
        (count1 == max_len).astype(jnp.float32),
    ])

    shp = (_ROWS, _COLS)
    ops = [x.reshape(shp) for x in (a0, b0, w0, r0, a1, b1, w1, r1, m)]

    grid = _ROWS // _BLOCK_ROWS
    blk = pl.BlockSpec((_BLOCK_ROWS, _COLS), lambda g: (g, 0))
    out = pl.pallas_call(
        _tail_body,
        grid=(grid,),
        in_specs=[pl.BlockSpec(memory_space=pltpu.SMEM)] + [blk] * 9,
        out_specs=pl.BlockSpec((1, 1), lambda g: (0, 0)),
        out_shape=jax.ShapeDtypeStruct((1, 1), jnp.float32),
    )(flags, *ops)

    reg_loss = out[0, 0]
    z = jnp.zeros((1,), dtype=jnp.float32)
    return (reg_loss, z, z, z)
